# baseline (speedup 1.0000x reference)
import functools

import jax
import jax.numpy as jnp
from jax import lax
from jax.experimental import pallas as pl
from jax.experimental.pallas import tpu as pltpu
from jax.experimental.pallas import tpu_sc as plsc

EMB = 128
NC = 2
NS = 16
NW = NC * NS
NBUF = 5
DEPTH = 3
CH = 80
N_PIECES = 4
ROWS_BLK = 8


def _emb_kernel_body(n_chunks, toks_per_w, x_hbm, tab_hbm, out_hbm, idx_v,
                     bufs, gsem, osem):
    wid = lax.axis_index("s") * NC + lax.axis_index("c")
    base = wid * toks_per_w
    pltpu.sync_copy(x_hbm.at[pl.ds(base, toks_per_w)], idx_v)

    def gather(c, slot):
        pltpu.async_copy(tab_hbm.at[idx_v.at[pl.ds(c * CH, CH)]],
                         bufs.at[slot], gsem)

    def out_slice(c):
        return out_hbm.at[pl.ds(base + c * CH, CH)]

    for c in range(DEPTH):
        gather(c, c)

    @pl.loop(0, n_chunks, step=NBUF)
    def _group(g):
        for b in range(NBUF):
            j = g + b

            @pl.when(j >= NBUF - DEPTH)
            def _():
                pltpu.make_async_copy(bufs.at[b], out_slice(j), osem).wait()

            pltpu.make_async_copy(tab_hbm.at[idx_v.at[pl.ds(0, CH)]],
                                  bufs.at[b], gsem).wait()
            pltpu.async_copy(bufs.at[b], out_slice(j), osem)

            @pl.when(j + DEPTH < n_chunks)
            def _():
                gather(j + DEPTH, (b + DEPTH) % NBUF)

    for b in range(NBUF - DEPTH):
        pltpu.make_async_copy(bufs.at[b], out_slice(0), osem).wait()


def _make_emb_call(piece_toks):
    toks_per_w = piece_toks // NW
    n_chunks = toks_per_w // CH
    mesh = plsc.VectorSubcoreMesh(core_axis_name="c", subcore_axis_name="s")
    return pl.kernel(
        functools.partial(_emb_kernel_body, n_chunks, toks_per_w),
        out_type=jax.ShapeDtypeStruct((piece_toks, EMB), jnp.float32),
        mesh=mesh,
        scratch_types=[
            pltpu.VMEM((toks_per_w,), jnp.int32),
            pltpu.VMEM((NBUF, CH, EMB), jnp.float32),
            pltpu.SemaphoreType.DMA,
            pltpu.SemaphoreType.DMA,
        ],
    )


def kernel(x, table):
    b, h = x.shape
    pb = b // N_PIECES
    sc_call = _make_emb_call(pb * h)
    xi = x.astype(jnp.int32).reshape(b * h)
    flats = [sc_call(xi[i * pb * h:(i + 1) * pb * h], table)
             for i in range(N_PIECES)]
    pieces = [f.reshape(pb, h, EMB) for f in flats]
    return jnp.stack(pieces, axis=0).reshape(b, h, EMB)


# speedup vs baseline: 2.1813x; 2.1813x over previous
import functools

import jax
import jax.numpy as jnp
from jax import lax
from jax.experimental import pallas as pl
from jax.experimental.pallas import tpu as pltpu
from jax.experimental.pallas import tpu_sc as plsc

EMB = 128
NC = 2
NS = 16
NW = NC * NS
NBUF = 4
DEPTH = 3
BR = 4


def _emb_kernel_body(n_chunks, rows_per_w, h, x_hbm, tab_hbm, out_hbm, idx_v,
                     bufs, gsem, osem):
    wid = lax.axis_index("s") * NC + lax.axis_index("c")
    base = wid * rows_per_w
    pltpu.sync_copy(x_hbm.at[pl.ds(base, rows_per_w)], idx_v)

    def gather(c, slot):
        for r in range(BR):
            pltpu.async_copy(tab_hbm.at[idx_v.at[c * BR + r]],
                             bufs.at[slot, r], gsem)

    def gather_wait(b):
        for r in range(BR):
            pltpu.make_async_copy(tab_hbm.at[idx_v.at[0]], bufs.at[b, r],
                                  gsem).wait()

    def out_slice(c):
        return out_hbm.at[pl.ds(base + c * BR, BR)]

    for c in range(DEPTH):
        gather(c, c)

    @pl.loop(0, n_chunks, step=NBUF)
    def _group(g):
        for b in range(NBUF):
            j = g + b

            @pl.when(j >= NBUF - DEPTH)
            def _():
                pltpu.make_async_copy(bufs.at[b], out_slice(j), osem).wait()

            gather_wait(b)
            pltpu.async_copy(bufs.at[b], out_slice(j), osem)

            @pl.when(j + DEPTH < n_chunks)
            def _():
                gather(j + DEPTH, (b + DEPTH) % NBUF)

    for b in range(NBUF - DEPTH):
        pltpu.make_async_copy(bufs.at[b], out_slice(0), osem).wait()


def _make_emb_call(bsz, h):
    rows_per_w = bsz // NW
    n_chunks = rows_per_w // BR
    mesh = plsc.VectorSubcoreMesh(core_axis_name="c", subcore_axis_name="s")
    return pl.kernel(
        functools.partial(_emb_kernel_body, n_chunks, rows_per_w, h),
        out_type=jax.ShapeDtypeStruct((bsz, h, EMB), jnp.float32),
        mesh=mesh,
        compiler_params=pltpu.CompilerParams(needs_layout_passes=True),
        scratch_types=[
            pltpu.VMEM((rows_per_w, h), jnp.int32),
            pltpu.VMEM((NBUF, BR, h, EMB), jnp.float32),
            pltpu.SemaphoreType.DMA,
            pltpu.SemaphoreType.DMA,
        ],
    )


def kernel(x, table):
    b, h = x.shape
    return _make_emb_call(b, h)(x.astype(jnp.int32), table)
